# baseline (speedup 1.0000x reference)
import dataclasses
import functools

import jax
import jax.numpy as jnp
from jax import lax
from jax.experimental import pallas as pl
from jax.experimental.pallas import tpu as pltpu
from jax.experimental.pallas import tpu_sc as plsc

N = 10000
E = 320000
NC_SC = 2
NS_SC = 16
NW = NC_SC * NS_SC
CHUNK = 128
NCHUNK = 80
EPT = CHUNK * NCHUNK
E_PAD = EPT * NW
NCHA = 40
NCHB = 120
EA = NS_SC * NCHA * CHUNK
NPAD = 10240
RPS = NPAD // NS_SC


def _sc_mesh():
    return plsc.VectorSubcoreMesh(core_axis_name="c", subcore_axis_name="s")


def _sc_params():
    cp = pltpu.CompilerParams()
    if "needs_layout_passes" in pltpu.CompilerParams.__dataclass_fields__:
        cp = dataclasses.replace(cp, needs_layout_passes=False)
    return cp


def _sc_degree(dstp3, wp3):
    @functools.partial(
        pl.kernel,
        out_type=jax.ShapeDtypeStruct((NC_SC, NPAD, 128), jnp.float32),
        mesh=_sc_mesh(),
        scratch_types=[
            pltpu.VMEM((CHUNK, 128), jnp.float32),
            pltpu.VMEM((CHUNK,), jnp.int32),
            pltpu.VMEM((CHUNK,), jnp.float32),
            pltpu.VMEM_SHARED((NPAD, 128), jnp.float32),
        ],
        compiler_params=_sc_params(),
    )
    def k(dst_hbm, w_hbm, out_hbm, wbuf, didx, wv, acc):
        cid = lax.axis_index("c")
        sid = lax.axis_index("s")
        wid = cid * NS_SC + sid
        lane0 = lax.iota(jnp.int32, 16) == 0

        @pl.loop(0, CHUNK)
        def _(r):
            for c in range(8):
                wbuf[r, pl.ds(c * 16, 16)] = jnp.zeros((16,), jnp.float32)

        @pl.loop(0, RPS // CHUNK)
        def _(j):
            pltpu.sync_copy(wbuf, acc.at[pl.ds(sid * RPS + j * CHUNK, CHUNK)])

        plsc.subcore_barrier()

        @pl.loop(0, NCHUNK)
        def _(i):
            pltpu.sync_copy(dst_hbm.at[wid, i], didx)
            pltpu.sync_copy(w_hbm.at[wid, i], wv)

            @pl.loop(0, CHUNK)
            def _(b):
                bvec = jnp.full((16,), b, jnp.int32)
                ws = plsc.load_gather(wv, [bvec])
                wbuf[b, pl.ds(0, 16)] = jnp.where(lane0, ws, 0.0)

            pltpu.sync_copy(wbuf, acc.at[didx], add=True)

        plsc.subcore_barrier()

        @pl.loop(0, RPS // CHUNK)
        def _(j):
            r0 = sid * RPS + j * CHUNK
            pltpu.sync_copy(acc.at[pl.ds(r0, CHUNK)],
                            out_hbm.at[cid, pl.ds(r0, CHUNK)])

    return k(dstp3, wp3)


def _sc_message(y, srcA, dstA, wA, srcB, dstB, wB, F):
    FC = F // 16

    @functools.partial(
        pl.kernel,
        out_type=jax.ShapeDtypeStruct((NC_SC, NPAD, F), jnp.float32),
        mesh=_sc_mesh(),
        scratch_types=[
            pltpu.VMEM((CHUNK, F), jnp.float32),
            pltpu.VMEM((CHUNK, F), jnp.float32),
            pltpu.VMEM((4, CHUNK), jnp.int32),
            pltpu.VMEM((4, CHUNK), jnp.int32),
            pltpu.VMEM((4 * CHUNK,), jnp.float32),
            pltpu.VMEM_SHARED((NPAD, F), jnp.float32),
            pltpu.SemaphoreType.DMA,
            pltpu.SemaphoreType.DMA,
            pltpu.SemaphoreType.DMA,
            pltpu.SemaphoreType.DMA,
            pltpu.SemaphoreType.DMA,
            pltpu.SemaphoreType.DMA,
            pltpu.SemaphoreType.DMA,
            pltpu.SemaphoreType.DMA,
        ],
        compiler_params=_sc_params(),
    )
    def k(y_hbm, srcA_hbm, dstA_hbm, wA_hbm, srcB_hbm, dstB_hbm, wB_hbm,
          out_hbm, rows0, rows1, sidx, didx, wv4, acc, gsem0, gsem1,
          ssem0, ssem1, isem0, isem1, isem2, isem3):
        cid = lax.axis_index("c")
        sid = lax.axis_index("s")

        @pl.loop(0, CHUNK)
        def _(r):
            for c in range(FC):
                rows0[r, pl.ds(c * 16, 16)] = jnp.zeros((16,), jnp.float32)

        @pl.loop(0, RPS // CHUNK)
        def _(j):
            pltpu.sync_copy(rows0, acc.at[pl.ds(sid * RPS + j * CHUNK, CHUNK)])

        plsc.subcore_barrier()

        isems = (isem0, isem1, isem2, isem3)

        def pipeline(nch, src_hbm, dst_hbm, w_hbm):
            def ifetch(i, p):
                pltpu.async_copy(src_hbm.at[sid, i], sidx.at[p], isems[p])
                pltpu.async_copy(dst_hbm.at[sid, i], didx.at[p], isems[p])
                pltpu.async_copy(w_hbm.at[sid, i],
                                 wv4.at[pl.ds(p * CHUNK, CHUNK)], isems[p])

            def iwait(i, p):
                pltpu.make_async_copy(
                    src_hbm.at[sid, i], sidx.at[p], isems[p]).wait()
                pltpu.make_async_copy(
                    dst_hbm.at[sid, i], didx.at[p], isems[p]).wait()
                pltpu.make_async_copy(
                    w_hbm.at[sid, i],
                    wv4.at[pl.ds(p * CHUNK, CHUNK)], isems[p]).wait()

            def gather(p, rows, sem):
                pltpu.async_copy(y_hbm.at[sidx.at[p]], rows, sem)

            def gwait(p, rows, sem):
                pltpu.make_async_copy(y_hbm.at[sidx.at[p]], rows, sem).wait()

            def scale(p, rows):
                @pl.loop(0, CHUNK, step=4)
                def _(b):
                    for u in range(4):
                        bvec = jnp.full((16,), p * CHUNK + b + u, jnp.int32)
                        ws = plsc.load_gather(wv4, [bvec])
                        for c in range(FC):
                            sl = pl.ds(c * 16, 16)
                            rows[b + u, sl] = rows[b + u, sl] * ws

            def scat(p, rows, sem):
                pltpu.async_copy(rows, acc.at[didx.at[p]], sem, add=True)

            def swait(p, rows, sem):
                pltpu.make_async_copy(rows, acc.at[didx.at[p]], sem).wait()

            for p in range(4):
                ifetch(p, p)
            for p in range(4):
                iwait(p, p)
            gather(0, rows0, gsem0)
            gather(1, rows1, gsem1)

            @pl.loop(0, nch - 4, step=4)
            def _(i):
                gwait(0, rows0, gsem0)
                scale(0, rows0)
                scat(0, rows0, ssem0)
                gwait(1, rows1, gsem1)
                scale(1, rows1)
                scat(1, rows1, ssem1)
                swait(0, rows0, ssem0)
                gather(2, rows0, gsem0)
                ifetch(i + 4, 0)
                swait(1, rows1, ssem1)
                gather(3, rows1, gsem1)
                ifetch(i + 5, 1)
                gwait(2, rows0, gsem0)
                scale(2, rows0)
                scat(2, rows0, ssem0)
                gwait(3, rows1, gsem1)
                scale(3, rows1)
                scat(3, rows1, ssem1)
                swait(2, rows0, ssem0)
                iwait(i + 4, 0)
                gather(0, rows0, gsem0)
                ifetch(i + 6, 2)
                swait(3, rows1, ssem1)
                iwait(i + 5, 1)
                gather(1, rows1, gsem1)
                ifetch(i + 7, 3)
                iwait(i + 6, 2)
                iwait(i + 7, 3)

            t = nch - 4
            gwait(0, rows0, gsem0)
            scale(0, rows0)
            scat(0, rows0, ssem0)
            gwait(1, rows1, gsem1)
            scale(1, rows1)
            scat(1, rows1, ssem1)
            swait(0, rows0, ssem0)
            gather(2, rows0, gsem0)
            swait(1, rows1, ssem1)
            gather(3, rows1, gsem1)
            gwait(2, rows0, gsem0)
            scale(2, rows0)
            scat(2, rows0, ssem0)
            gwait(3, rows1, gsem1)
            scale(3, rows1)
            scat(3, rows1, ssem1)
            swait(2, rows0, ssem0)
            swait(3, rows1, ssem1)

        @pl.when(cid == 0)
        def _():
            pipeline(NCHA, srcA_hbm, dstA_hbm, wA_hbm)

        @pl.when(cid == 1)
        def _():
            pipeline(NCHB, srcB_hbm, dstB_hbm, wB_hbm)

        plsc.subcore_barrier()

        @pl.loop(0, RPS // CHUNK)
        def _(j):
            r0 = sid * RPS + j * CHUNK
            pltpu.sync_copy(acc.at[pl.ds(r0, CHUNK)],
                            out_hbm.at[cid, pl.ds(r0, CHUNK)])

    return k(y, srcA, dstA, wA, srcB, dstB, wB)


def _tc_prologue(x, WiT, bi, W1T, degp):
    def f(x_ref, wit_ref, bi_ref, w1t_ref, deg_ref, y1_ref, dis_ref):
        h0 = jnp.maximum(
            jnp.dot(x_ref[...], wit_ref[...],
                    preferred_element_type=jnp.float32) + bi_ref[...], 0.0)
        deg = deg_ref[0, :N, 0] + deg_ref[1, :N, 0] + 1.0
        dis = lax.rsqrt(deg)
        xw = jnp.dot(h0, w1t_ref[...], preferred_element_type=jnp.float32)
        y1_ref[...] = xw * dis[:, None]
        dis_ref[...] = dis

    return pl.pallas_call(
        f,
        out_shape=(
            jax.ShapeDtypeStruct((N, 128), jnp.float32),
            jax.ShapeDtypeStruct((N,), jnp.float32),
        ),
    )(x, WiT, bi, W1T, degp)


def _tc_mid(accp, y, dis, b, g, be, WnT, Fn):
    def f(acc_ref, y_ref, dis_ref, b_ref, g_ref, be_ref, wnt_ref, yn_ref):
        dis = dis_ref[...]
        t = (acc_ref[0, :N, :] + acc_ref[1, :N, :] + y_ref[...])
        t = t * dis[:, None] + b_ref[...]
        m = jnp.mean(t, axis=0)
        v = jnp.mean((t - m[None, :]) ** 2, axis=0)
        h = (t - m[None, :]) * lax.rsqrt(v[None, :] + 1e-5)
        h = jnp.maximum(h * g_ref[...] + be_ref[...], 0.0)
        xw = jnp.dot(h, wnt_ref[...], preferred_element_type=jnp.float32)
        yn = xw * dis[:, None]
        if Fn < 128:
            yn = jnp.concatenate(
                [yn, jnp.zeros((N, 128 - Fn), jnp.float32)], axis=1)
        yn_ref[...] = yn

    return pl.pallas_call(
        f,
        out_shape=jax.ShapeDtypeStruct((N, 128), jnp.float32),
    )(accp, y, dis, b, g, be, WnT)


def _tc_epilogue(accp, y, dis, b3, g3, be3, WoT, bo):
    def f(acc_ref, y_ref, dis_ref, b_ref, g_ref, be_ref, wot_ref, bo_ref,
          out_ref):
        dis = dis_ref[...]
        t = (acc_ref[0, :N, :64] + acc_ref[1, :N, :64] + y_ref[:, :64])
        t = t * dis[:, None] + b_ref[...]
        m = jnp.mean(t, axis=0)
        v = jnp.mean((t - m[None, :]) ** 2, axis=0)
        h = (t - m[None, :]) * lax.rsqrt(v[None, :] + 1e-5)
        h = jnp.maximum(h * g_ref[...] + be_ref[...], 0.0)
        logits = jnp.dot(h, wot_ref[...],
                         preferred_element_type=jnp.float32) + bo_ref[...]
        mx = jnp.max(logits, axis=1, keepdims=True)
        s = logits - mx
        lse = jnp.log(jnp.sum(jnp.exp(s), axis=1, keepdims=True))
        out_ref[...] = s - lse

    return pl.pallas_call(
        f,
        out_shape=jax.ShapeDtypeStruct((N, 10), jnp.float32),
    )(accp, y, dis, b3, g3, be3, WoT, bo)


def kernel(x, edge_index, edge_attr, Wi, bi, W1, b1, g1, be1, W2, b2, g2, be2,
           W3, b3, g3, be3, Wo, bo):
    src = edge_index[0].astype(jnp.int32)
    dst = edge_index[1].astype(jnp.int32)
    w = edge_attr.astype(jnp.float32)

    pad = E_PAD - E
    srcp = jnp.concatenate([src, jnp.zeros((pad,), jnp.int32)])
    dstp = jnp.concatenate([dst, jnp.zeros((pad,), jnp.int32)])
    wp = jnp.concatenate([w, jnp.zeros((pad,), jnp.float32)])
    dstp3 = dstp.reshape(NW, NCHUNK, CHUNK)
    wp3 = wp.reshape(NW, NCHUNK, CHUNK)

    srcA = srcp[:EA].reshape(NS_SC, NCHA, CHUNK)
    dstA = dstp[:EA].reshape(NS_SC, NCHA, CHUNK)
    wA = wp[:EA].reshape(NS_SC, NCHA, CHUNK)
    srcB = srcp[EA:].reshape(NS_SC, NCHB, CHUNK)
    dstB = dstp[EA:].reshape(NS_SC, NCHB, CHUNK)
    wB = wp[EA:].reshape(NS_SC, NCHB, CHUNK)
    edges = (srcA, dstA, wA, srcB, dstB, wB)

    degp = _sc_degree(dstp3, wp3)

    y1, dis = _tc_prologue(x, Wi.T, bi, W1.T, degp)

    acc1 = _sc_message(y1, *edges, 128)
    y2 = _tc_mid(acc1, y1, dis, b1, g1, be1, W2.T, 128)

    acc2 = _sc_message(y2, *edges, 128)
    y3 = _tc_mid(acc2, y2, dis, b2, g2, be2, W3.T, 64)

    acc3 = _sc_message(y3, *edges, 128)
    out = _tc_epilogue(acc3, y3, dis, b3, g3, be3, Wo.T, bo)
    return out


# speedup vs baseline: 1.2480x; 1.2480x over previous
import dataclasses
import functools

import jax
import jax.numpy as jnp
from jax import lax
from jax.experimental import pallas as pl
from jax.experimental.pallas import tpu as pltpu
from jax.experimental.pallas import tpu_sc as plsc

N = 10000
E = 320000
NC_SC = 2
NS_SC = 16
NW = NC_SC * NS_SC
CHUNK = 128
NCHUNK = 80
EPT = CHUNK * NCHUNK
E_PAD = EPT * NW
NCHA = 120
NCHB = 40
EA = NS_SC * NCHA * CHUNK
NPAD = 10240
RPS = NPAD // NS_SC


def _sc_mesh():
    return plsc.VectorSubcoreMesh(core_axis_name="c", subcore_axis_name="s")


def _sc_params():
    cp = pltpu.CompilerParams()
    if "needs_layout_passes" in pltpu.CompilerParams.__dataclass_fields__:
        cp = dataclasses.replace(cp, needs_layout_passes=False)
    return cp


def _sc_degree(dstp3, wp3):
    @functools.partial(
        pl.kernel,
        out_type=jax.ShapeDtypeStruct((NC_SC, NPAD, 128), jnp.float32),
        mesh=_sc_mesh(),
        scratch_types=[
            pltpu.VMEM((CHUNK, 128), jnp.float32),
            pltpu.VMEM((CHUNK,), jnp.int32),
            pltpu.VMEM((CHUNK,), jnp.float32),
            pltpu.VMEM_SHARED((NPAD, 128), jnp.float32),
        ],
        compiler_params=_sc_params(),
    )
    def k(dst_hbm, w_hbm, out_hbm, wbuf, didx, wv, acc):
        cid = lax.axis_index("c")
        sid = lax.axis_index("s")
        wid = cid * NS_SC + sid
        lane0 = lax.iota(jnp.int32, 16) == 0

        @pl.loop(0, CHUNK)
        def _(r):
            for c in range(8):
                wbuf[r, pl.ds(c * 16, 16)] = jnp.zeros((16,), jnp.float32)

        @pl.loop(0, RPS // CHUNK)
        def _(j):
            pltpu.sync_copy(wbuf, acc.at[pl.ds(sid * RPS + j * CHUNK, CHUNK)])

        plsc.subcore_barrier()

        @pl.loop(0, NCHUNK)
        def _(i):
            pltpu.sync_copy(dst_hbm.at[wid, i], didx)
            pltpu.sync_copy(w_hbm.at[wid, i], wv)

            @pl.loop(0, CHUNK)
            def _(b):
                bvec = jnp.full((16,), b, jnp.int32)
                ws = plsc.load_gather(wv, [bvec])
                wbuf[b, pl.ds(0, 16)] = jnp.where(lane0, ws, 0.0)

            pltpu.sync_copy(wbuf, acc.at[didx], add=True)

        plsc.subcore_barrier()

        @pl.loop(0, RPS // CHUNK)
        def _(j):
            r0 = sid * RPS + j * CHUNK
            pltpu.sync_copy(acc.at[pl.ds(r0, CHUNK)],
                            out_hbm.at[cid, pl.ds(r0, CHUNK)])

    return k(dstp3, wp3)


def _sc_message(y, srcA, dstA, wA, srcB, dstB, wB, F):
    FC = F // 16

    @functools.partial(
        pl.kernel,
        out_type=jax.ShapeDtypeStruct((NC_SC, NPAD, F), jnp.float32),
        mesh=_sc_mesh(),
        scratch_types=[
            pltpu.VMEM((CHUNK, F), jnp.float32),
            pltpu.VMEM((CHUNK, F), jnp.float32),
            pltpu.VMEM((4, CHUNK), jnp.int32),
            pltpu.VMEM((4, CHUNK), jnp.int32),
            pltpu.VMEM((4 * CHUNK,), jnp.float32),
            pltpu.VMEM_SHARED((NPAD, F), jnp.float32),
            pltpu.SemaphoreType.DMA,
            pltpu.SemaphoreType.DMA,
            pltpu.SemaphoreType.DMA,
            pltpu.SemaphoreType.DMA,
            pltpu.SemaphoreType.DMA,
            pltpu.SemaphoreType.DMA,
            pltpu.SemaphoreType.DMA,
            pltpu.SemaphoreType.DMA,
        ],
        compiler_params=_sc_params(),
    )
    def k(y_hbm, srcA_hbm, dstA_hbm, wA_hbm, srcB_hbm, dstB_hbm, wB_hbm,
          out_hbm, rows0, rows1, sidx, didx, wv4, acc, gsem0, gsem1,
          ssem0, ssem1, isem0, isem1, isem2, isem3):
        cid = lax.axis_index("c")
        sid = lax.axis_index("s")

        @pl.loop(0, CHUNK)
        def _(r):
            for c in range(FC):
                rows0[r, pl.ds(c * 16, 16)] = jnp.zeros((16,), jnp.float32)

        @pl.loop(0, RPS // CHUNK)
        def _(j):
            pltpu.sync_copy(rows0, acc.at[pl.ds(sid * RPS + j * CHUNK, CHUNK)])

        plsc.subcore_barrier()

        isems = (isem0, isem1, isem2, isem3)

        def pipeline(nch, src_hbm, dst_hbm, w_hbm):
            def ifetch(i, p):
                pltpu.async_copy(src_hbm.at[sid, i], sidx.at[p], isems[p])
                pltpu.async_copy(dst_hbm.at[sid, i], didx.at[p], isems[p])
                pltpu.async_copy(w_hbm.at[sid, i],
                                 wv4.at[pl.ds(p * CHUNK, CHUNK)], isems[p])

            def iwait(i, p):
                pltpu.make_async_copy(
                    src_hbm.at[sid, i], sidx.at[p], isems[p]).wait()
                pltpu.make_async_copy(
                    dst_hbm.at[sid, i], didx.at[p], isems[p]).wait()
                pltpu.make_async_copy(
                    w_hbm.at[sid, i],
                    wv4.at[pl.ds(p * CHUNK, CHUNK)], isems[p]).wait()

            def gather(p, rows, sem):
                pltpu.async_copy(y_hbm.at[sidx.at[p]], rows, sem)

            def gwait(p, rows, sem):
                pltpu.make_async_copy(y_hbm.at[sidx.at[p]], rows, sem).wait()

            def scale(p, rows):
                @pl.loop(0, CHUNK, step=4)
                def _(b):
                    for u in range(4):
                        bvec = jnp.full((16,), p * CHUNK + b + u, jnp.int32)
                        ws = plsc.load_gather(wv4, [bvec])
                        for c in range(FC):
                            sl = pl.ds(c * 16, 16)
                            rows[b + u, sl] = rows[b + u, sl] * ws

            def scat(p, rows, sem):
                pltpu.async_copy(rows, acc.at[didx.at[p]], sem, add=True)

            def swait(p, rows, sem):
                pltpu.make_async_copy(rows, acc.at[didx.at[p]], sem).wait()

            for p in range(4):
                ifetch(p, p)
            for p in range(4):
                iwait(p, p)
            gather(0, rows0, gsem0)
            gather(1, rows1, gsem1)

            @pl.loop(0, nch - 4, step=4)
            def _(i):
                gwait(0, rows0, gsem0)
                scale(0, rows0)
                scat(0, rows0, ssem0)
                gwait(1, rows1, gsem1)
                scale(1, rows1)
                scat(1, rows1, ssem1)
                swait(0, rows0, ssem0)
                gather(2, rows0, gsem0)
                ifetch(i + 4, 0)
                swait(1, rows1, ssem1)
                gather(3, rows1, gsem1)
                ifetch(i + 5, 1)
                gwait(2, rows0, gsem0)
                scale(2, rows0)
                scat(2, rows0, ssem0)
                gwait(3, rows1, gsem1)
                scale(3, rows1)
                scat(3, rows1, ssem1)
                swait(2, rows0, ssem0)
                iwait(i + 4, 0)
                gather(0, rows0, gsem0)
                ifetch(i + 6, 2)
                swait(3, rows1, ssem1)
                iwait(i + 5, 1)
                gather(1, rows1, gsem1)
                ifetch(i + 7, 3)
                iwait(i + 6, 2)
                iwait(i + 7, 3)

            t = nch - 4
            gwait(0, rows0, gsem0)
            scale(0, rows0)
            scat(0, rows0, ssem0)
            gwait(1, rows1, gsem1)
            scale(1, rows1)
            scat(1, rows1, ssem1)
            swait(0, rows0, ssem0)
            gather(2, rows0, gsem0)
            swait(1, rows1, ssem1)
            gather(3, rows1, gsem1)
            gwait(2, rows0, gsem0)
            scale(2, rows0)
            scat(2, rows0, ssem0)
            gwait(3, rows1, gsem1)
            scale(3, rows1)
            scat(3, rows1, ssem1)
            swait(2, rows0, ssem0)
            swait(3, rows1, ssem1)

        @pl.when(cid == 0)
        def _():
            pipeline(NCHA, srcA_hbm, dstA_hbm, wA_hbm)

        @pl.when(cid == 1)
        def _():
            pipeline(NCHB, srcB_hbm, dstB_hbm, wB_hbm)

        plsc.subcore_barrier()

        @pl.loop(0, RPS // CHUNK)
        def _(j):
            r0 = sid * RPS + j * CHUNK
            pltpu.sync_copy(acc.at[pl.ds(r0, CHUNK)],
                            out_hbm.at[cid, pl.ds(r0, CHUNK)])

    return k(y, srcA, dstA, wA, srcB, dstB, wB)


def _tc_prologue(x, WiT, bi, W1T, degp):
    def f(x_ref, wit_ref, bi_ref, w1t_ref, deg_ref, y1_ref, dis_ref):
        h0 = jnp.maximum(
            jnp.dot(x_ref[...], wit_ref[...],
                    preferred_element_type=jnp.float32) + bi_ref[...], 0.0)
        deg = deg_ref[0, :N, 0] + deg_ref[1, :N, 0] + 1.0
        dis = lax.rsqrt(deg)
        xw = jnp.dot(h0, w1t_ref[...], preferred_element_type=jnp.float32)
        y1_ref[...] = xw * dis[:, None]
        dis_ref[...] = dis

    return pl.pallas_call(
        f,
        out_shape=(
            jax.ShapeDtypeStruct((N, 128), jnp.float32),
            jax.ShapeDtypeStruct((N,), jnp.float32),
        ),
    )(x, WiT, bi, W1T, degp)


def _tc_mid(accp, y, dis, b, g, be, WnT, Fn):
    def f(acc_ref, y_ref, dis_ref, b_ref, g_ref, be_ref, wnt_ref, yn_ref):
        dis = dis_ref[...]
        t = (acc_ref[0, :N, :] + acc_ref[1, :N, :] + y_ref[...])
        t = t * dis[:, None] + b_ref[...]
        m = jnp.mean(t, axis=0)
        v = jnp.mean((t - m[None, :]) ** 2, axis=0)
        h = (t - m[None, :]) * lax.rsqrt(v[None, :] + 1e-5)
        h = jnp.maximum(h * g_ref[...] + be_ref[...], 0.0)
        xw = jnp.dot(h, wnt_ref[...], preferred_element_type=jnp.float32)
        yn = xw * dis[:, None]
        if Fn < 128:
            yn = jnp.concatenate(
                [yn, jnp.zeros((N, 128 - Fn), jnp.float32)], axis=1)
        yn_ref[...] = yn

    return pl.pallas_call(
        f,
        out_shape=jax.ShapeDtypeStruct((N, 128), jnp.float32),
    )(accp, y, dis, b, g, be, WnT)


def _tc_epilogue(accp, y, dis, b3, g3, be3, WoT, bo):
    def f(acc_ref, y_ref, dis_ref, b_ref, g_ref, be_ref, wot_ref, bo_ref,
          out_ref):
        dis = dis_ref[...]
        t = (acc_ref[0, :N, :64] + acc_ref[1, :N, :64] + y_ref[:, :64])
        t = t * dis[:, None] + b_ref[...]
        m = jnp.mean(t, axis=0)
        v = jnp.mean((t - m[None, :]) ** 2, axis=0)
        h = (t - m[None, :]) * lax.rsqrt(v[None, :] + 1e-5)
        h = jnp.maximum(h * g_ref[...] + be_ref[...], 0.0)
        logits = jnp.dot(h, wot_ref[...],
                         preferred_element_type=jnp.float32) + bo_ref[...]
        mx = jnp.max(logits, axis=1, keepdims=True)
        s = logits - mx
        lse = jnp.log(jnp.sum(jnp.exp(s), axis=1, keepdims=True))
        out_ref[...] = s - lse

    return pl.pallas_call(
        f,
        out_shape=jax.ShapeDtypeStruct((N, 10), jnp.float32),
    )(accp, y, dis, b3, g3, be3, WoT, bo)


def kernel(x, edge_index, edge_attr, Wi, bi, W1, b1, g1, be1, W2, b2, g2, be2,
           W3, b3, g3, be3, Wo, bo):
    src = edge_index[0].astype(jnp.int32)
    dst = edge_index[1].astype(jnp.int32)
    w = edge_attr.astype(jnp.float32)

    pad = E_PAD - E
    srcp = jnp.concatenate([src, jnp.zeros((pad,), jnp.int32)])
    dstp = jnp.concatenate([dst, jnp.zeros((pad,), jnp.int32)])
    wp = jnp.concatenate([w, jnp.zeros((pad,), jnp.float32)])
    dstp3 = dstp.reshape(NW, NCHUNK, CHUNK)
    wp3 = wp.reshape(NW, NCHUNK, CHUNK)

    srcA = srcp[:EA].reshape(NS_SC, NCHA, CHUNK)
    dstA = dstp[:EA].reshape(NS_SC, NCHA, CHUNK)
    wA = wp[:EA].reshape(NS_SC, NCHA, CHUNK)
    srcB = srcp[EA:].reshape(NS_SC, NCHB, CHUNK)
    dstB = dstp[EA:].reshape(NS_SC, NCHB, CHUNK)
    wB = wp[EA:].reshape(NS_SC, NCHB, CHUNK)
    edges = (srcA, dstA, wA, srcB, dstB, wB)

    degp = _sc_degree(dstp3, wp3)

    y1, dis = _tc_prologue(x, Wi.T, bi, W1.T, degp)

    acc1 = _sc_message(y1, *edges, 128)
    y2 = _tc_mid(acc1, y1, dis, b1, g1, be1, W2.T, 128)

    acc2 = _sc_message(y2, *edges, 128)
    y3 = _tc_mid(acc2, y2, dis, b2, g2, be2, W3.T, 64)

    acc3 = _sc_message(y3, *edges, 128)
    out = _tc_epilogue(acc3, y3, dis, b3, g3, be3, Wo.T, bo)
    return out


# speedup vs baseline: 1.2494x; 1.0011x over previous
import dataclasses
import functools

import jax
import jax.numpy as jnp
from jax import lax
from jax.experimental import pallas as pl
from jax.experimental.pallas import tpu as pltpu
from jax.experimental.pallas import tpu_sc as plsc

N = 10000
E = 320000
NC_SC = 2
NS_SC = 16
NW = NC_SC * NS_SC
CHUNK = 128
NCHUNK = 80
EPT = CHUNK * NCHUNK
E_PAD = EPT * NW
NCHA = 120
NCHB = 40
EA = NS_SC * NCHA * CHUNK
NPAD = 10240
RPS = NPAD // NS_SC


def _sc_mesh():
    return plsc.VectorSubcoreMesh(core_axis_name="c", subcore_axis_name="s")


def _sc_params():
    cp = pltpu.CompilerParams()
    if "needs_layout_passes" in pltpu.CompilerParams.__dataclass_fields__:
        cp = dataclasses.replace(cp, needs_layout_passes=False)
    return cp


def _sc_degree(dstp3, wp3):
    @functools.partial(
        pl.kernel,
        out_type=jax.ShapeDtypeStruct((NC_SC, NPAD, 128), jnp.float32),
        mesh=_sc_mesh(),
        scratch_types=[
            pltpu.VMEM((CHUNK, 128), jnp.float32),
            pltpu.VMEM((CHUNK,), jnp.int32),
            pltpu.VMEM((CHUNK,), jnp.float32),
            pltpu.VMEM_SHARED((NPAD, 128), jnp.float32),
        ],
        compiler_params=_sc_params(),
    )
    def k(dst_hbm, w_hbm, out_hbm, wbuf, didx, wv, acc):
        cid = lax.axis_index("c")
        sid = lax.axis_index("s")
        wid = cid * NS_SC + sid
        lane0 = lax.iota(jnp.int32, 16) == 0

        @pl.loop(0, CHUNK)
        def _(r):
            for c in range(8):
                wbuf[r, pl.ds(c * 16, 16)] = jnp.zeros((16,), jnp.float32)

        @pl.loop(0, RPS // CHUNK)
        def _(j):
            pltpu.sync_copy(wbuf, acc.at[pl.ds(sid * RPS + j * CHUNK, CHUNK)])

        plsc.subcore_barrier()

        @pl.loop(0, NCHUNK)
        def _(i):
            pltpu.sync_copy(dst_hbm.at[wid, i], didx)
            pltpu.sync_copy(w_hbm.at[wid, i], wv)

            @pl.loop(0, CHUNK)
            def _(b):
                bvec = jnp.full((16,), b, jnp.int32)
                ws = plsc.load_gather(wv, [bvec])
                wbuf[b, pl.ds(0, 16)] = jnp.where(lane0, ws, 0.0)

            pltpu.sync_copy(wbuf, acc.at[didx], add=True)

        plsc.subcore_barrier()

        @pl.loop(0, RPS // CHUNK)
        def _(j):
            r0 = sid * RPS + j * CHUNK
            pltpu.sync_copy(acc.at[pl.ds(r0, CHUNK)],
                            out_hbm.at[cid, pl.ds(r0, CHUNK)])

    return k(dstp3, wp3)


def _sc_message(y, srcA, dstA, wA, srcB, dstB, wB, F):
    FC = F // 16

    @functools.partial(
        pl.kernel,
        out_type=jax.ShapeDtypeStruct((NC_SC, NPAD, F), jnp.float32),
        mesh=_sc_mesh(),
        scratch_types=[
            pltpu.VMEM((CHUNK, F), jnp.float32),
            pltpu.VMEM((CHUNK, F), jnp.float32),
            pltpu.VMEM((4, CHUNK), jnp.int32),
            pltpu.VMEM((4, CHUNK), jnp.int32),
            pltpu.VMEM((4 * CHUNK,), jnp.float32),
            pltpu.VMEM_SHARED((NPAD, F), jnp.float32),
            pltpu.SemaphoreType.DMA,
            pltpu.SemaphoreType.DMA,
            pltpu.SemaphoreType.DMA,
            pltpu.SemaphoreType.DMA,
            pltpu.SemaphoreType.DMA,
            pltpu.SemaphoreType.DMA,
            pltpu.SemaphoreType.DMA,
            pltpu.SemaphoreType.DMA,
        ],
        compiler_params=_sc_params(),
    )
    def k(y_hbm, srcA_hbm, dstA_hbm, wA_hbm, srcB_hbm, dstB_hbm, wB_hbm,
          out_hbm, rows0, rows1, sidx, didx, wv4, acc, gsem0, gsem1,
          ssem0, ssem1, isem0, isem1, isem2, isem3):
        cid = lax.axis_index("c")
        sid = lax.axis_index("s")

        @pl.loop(0, CHUNK)
        def _(r):
            for c in range(FC):
                rows0[r, pl.ds(c * 16, 16)] = jnp.zeros((16,), jnp.float32)

        @pl.loop(0, RPS // CHUNK)
        def _(j):
            pltpu.sync_copy(rows0, acc.at[pl.ds(sid * RPS + j * CHUNK, CHUNK)])

        plsc.subcore_barrier()

        isems = (isem0, isem1, isem2, isem3)

        def pipeline(nch, src_hbm, dst_hbm, w_hbm):
            def ifetch(i, p):
                pltpu.async_copy(src_hbm.at[sid, i], sidx.at[p], isems[p])
                pltpu.async_copy(dst_hbm.at[sid, i], didx.at[p], isems[p])
                pltpu.async_copy(w_hbm.at[sid, i],
                                 wv4.at[pl.ds(p * CHUNK, CHUNK)], isems[p])

            def iwait(i, p):
                pltpu.make_async_copy(
                    src_hbm.at[sid, i], sidx.at[p], isems[p]).wait()
                pltpu.make_async_copy(
                    dst_hbm.at[sid, i], didx.at[p], isems[p]).wait()
                pltpu.make_async_copy(
                    w_hbm.at[sid, i],
                    wv4.at[pl.ds(p * CHUNK, CHUNK)], isems[p]).wait()

            def gather(p, rows, sem):
                pltpu.async_copy(y_hbm.at[sidx.at[p]], rows, sem)

            def gwait(p, rows, sem):
                pltpu.make_async_copy(y_hbm.at[sidx.at[p]], rows, sem).wait()

            def scale(p, rows):
                @pl.loop(0, CHUNK)
                def _(b):
                    bvec = jnp.full((16,), p * CHUNK + b, jnp.int32)
                    ws = plsc.load_gather(wv4, [bvec])
                    for c in range(FC):
                        sl = pl.ds(c * 16, 16)
                        rows[b, sl] = rows[b, sl] * ws

            def scat(p, rows, sem):
                pltpu.async_copy(rows, acc.at[didx.at[p]], sem, add=True)

            def swait(p, rows, sem):
                pltpu.make_async_copy(rows, acc.at[didx.at[p]], sem).wait()

            for p in range(4):
                ifetch(p, p)
            for p in range(4):
                iwait(p, p)
            gather(0, rows0, gsem0)
            gather(1, rows1, gsem1)

            @pl.loop(0, nch - 4, step=4)
            def _(i):
                gwait(0, rows0, gsem0)
                scale(0, rows0)
                scat(0, rows0, ssem0)
                gwait(1, rows1, gsem1)
                scale(1, rows1)
                scat(1, rows1, ssem1)
                swait(0, rows0, ssem0)
                gather(2, rows0, gsem0)
                ifetch(i + 4, 0)
                swait(1, rows1, ssem1)
                gather(3, rows1, gsem1)
                ifetch(i + 5, 1)
                gwait(2, rows0, gsem0)
                scale(2, rows0)
                scat(2, rows0, ssem0)
                gwait(3, rows1, gsem1)
                scale(3, rows1)
                scat(3, rows1, ssem1)
                swait(2, rows0, ssem0)
                iwait(i + 4, 0)
                gather(0, rows0, gsem0)
                ifetch(i + 6, 2)
                swait(3, rows1, ssem1)
                iwait(i + 5, 1)
                gather(1, rows1, gsem1)
                ifetch(i + 7, 3)
                iwait(i + 6, 2)
                iwait(i + 7, 3)

            t = nch - 4
            gwait(0, rows0, gsem0)
            scale(0, rows0)
            scat(0, rows0, ssem0)
            gwait(1, rows1, gsem1)
            scale(1, rows1)
            scat(1, rows1, ssem1)
            swait(0, rows0, ssem0)
            gather(2, rows0, gsem0)
            swait(1, rows1, ssem1)
            gather(3, rows1, gsem1)
            gwait(2, rows0, gsem0)
            scale(2, rows0)
            scat(2, rows0, ssem0)
            gwait(3, rows1, gsem1)
            scale(3, rows1)
            scat(3, rows1, ssem1)
            swait(2, rows0, ssem0)
            swait(3, rows1, ssem1)

        @pl.when(cid == 0)
        def _():
            pipeline(NCHA, srcA_hbm, dstA_hbm, wA_hbm)

        @pl.when(cid == 1)
        def _():
            pipeline(NCHB, srcB_hbm, dstB_hbm, wB_hbm)

        plsc.subcore_barrier()

        @pl.loop(0, RPS // CHUNK)
        def _(j):
            r0 = sid * RPS + j * CHUNK
            pltpu.sync_copy(acc.at[pl.ds(r0, CHUNK)],
                            out_hbm.at[cid, pl.ds(r0, CHUNK)])

    return k(y, srcA, dstA, wA, srcB, dstB, wB)


def _tc_prologue(x, WiT, bi, W1T, degp):
    def f(x_ref, wit_ref, bi_ref, w1t_ref, deg_ref, y1_ref, dis_ref):
        h0 = jnp.maximum(
            jnp.dot(x_ref[...], wit_ref[...],
                    preferred_element_type=jnp.float32) + bi_ref[...], 0.0)
        deg = deg_ref[0, :N, 0] + deg_ref[1, :N, 0] + 1.0
        dis = lax.rsqrt(deg)
        xw = jnp.dot(h0, w1t_ref[...], preferred_element_type=jnp.float32)
        y1_ref[...] = xw * dis[:, None]
        dis_ref[...] = dis

    return pl.pallas_call(
        f,
        out_shape=(
            jax.ShapeDtypeStruct((N, 128), jnp.float32),
            jax.ShapeDtypeStruct((N,), jnp.float32),
        ),
    )(x, WiT, bi, W1T, degp)


def _tc_mid(accp, y, dis, b, g, be, WnT, Fn):
    def f(acc_ref, y_ref, dis_ref, b_ref, g_ref, be_ref, wnt_ref, yn_ref):
        dis = dis_ref[...]
        t = (acc_ref[0, :N, :] + acc_ref[1, :N, :] + y_ref[...])
        t = t * dis[:, None] + b_ref[...]
        m = jnp.mean(t, axis=0)
        v = jnp.mean((t - m[None, :]) ** 2, axis=0)
        h = (t - m[None, :]) * lax.rsqrt(v[None, :] + 1e-5)
        h = jnp.maximum(h * g_ref[...] + be_ref[...], 0.0)
        xw = jnp.dot(h, wnt_ref[...], preferred_element_type=jnp.float32)
        yn = xw * dis[:, None]
        if Fn < 128:
            yn = jnp.concatenate(
                [yn, jnp.zeros((N, 128 - Fn), jnp.float32)], axis=1)
        yn_ref[...] = yn

    return pl.pallas_call(
        f,
        out_shape=jax.ShapeDtypeStruct((N, 128), jnp.float32),
    )(accp, y, dis, b, g, be, WnT)


def _tc_epilogue(accp, y, dis, b3, g3, be3, WoT, bo):
    def f(acc_ref, y_ref, dis_ref, b_ref, g_ref, be_ref, wot_ref, bo_ref,
          out_ref):
        dis = dis_ref[...]
        t = (acc_ref[0, :N, :64] + acc_ref[1, :N, :64] + y_ref[:, :64])
        t = t * dis[:, None] + b_ref[...]
        m = jnp.mean(t, axis=0)
        v = jnp.mean((t - m[None, :]) ** 2, axis=0)
        h = (t - m[None, :]) * lax.rsqrt(v[None, :] + 1e-5)
        h = jnp.maximum(h * g_ref[...] + be_ref[...], 0.0)
        logits = jnp.dot(h, wot_ref[...],
                         preferred_element_type=jnp.float32) + bo_ref[...]
        mx = jnp.max(logits, axis=1, keepdims=True)
        s = logits - mx
        lse = jnp.log(jnp.sum(jnp.exp(s), axis=1, keepdims=True))
        out_ref[...] = s - lse

    return pl.pallas_call(
        f,
        out_shape=jax.ShapeDtypeStruct((N, 10), jnp.float32),
    )(accp, y, dis, b3, g3, be3, WoT, bo)


def kernel(x, edge_index, edge_attr, Wi, bi, W1, b1, g1, be1, W2, b2, g2, be2,
           W3, b3, g3, be3, Wo, bo):
    src = edge_index[0].astype(jnp.int32)
    dst = edge_index[1].astype(jnp.int32)
    w = edge_attr.astype(jnp.float32)

    pad = E_PAD - E
    srcp = jnp.concatenate([src, jnp.zeros((pad,), jnp.int32)])
    dstp = jnp.concatenate([dst, jnp.zeros((pad,), jnp.int32)])
    wp = jnp.concatenate([w, jnp.zeros((pad,), jnp.float32)])
    dstp3 = dstp.reshape(NW, NCHUNK, CHUNK)
    wp3 = wp.reshape(NW, NCHUNK, CHUNK)

    srcA = srcp[:EA].reshape(NS_SC, NCHA, CHUNK)
    dstA = dstp[:EA].reshape(NS_SC, NCHA, CHUNK)
    wA = wp[:EA].reshape(NS_SC, NCHA, CHUNK)
    srcB = srcp[EA:].reshape(NS_SC, NCHB, CHUNK)
    dstB = dstp[EA:].reshape(NS_SC, NCHB, CHUNK)
    wB = wp[EA:].reshape(NS_SC, NCHB, CHUNK)
    edges = (srcA, dstA, wA, srcB, dstB, wB)

    degp = _sc_degree(dstp3, wp3)

    y1, dis = _tc_prologue(x, Wi.T, bi, W1.T, degp)

    acc1 = _sc_message(y1, *edges, 128)
    y2 = _tc_mid(acc1, y1, dis, b1, g1, be1, W2.T, 128)

    acc2 = _sc_message(y2, *edges, 128)
    y3 = _tc_mid(acc2, y2, dis, b2, g2, be2, W3.T, 64)

    acc3 = _sc_message(y3, *edges, 128)
    out = _tc_epilogue(acc3, y3, dis, b3, g3, be3, Wo.T, bo)
    return out
